# baseline (speedup 1.0000x reference)
import functools

import jax
import jax.numpy as jnp
from jax import lax
from jax.experimental import pallas as pl
from jax.experimental.pallas import tpu as pltpu
from jax.experimental.pallas import tpu_sc as plsc

NH = 32
N = 4096
E = 256
B = 2
NKEEP = 2048
ROWS = B * N
DUMP = B * NKEEP


def _gm_body(x_ref, gm_ref):
    s = jnp.sum(x_ref[...], axis=1, keepdims=True)
    gm_ref[...] = jnp.abs(s / jnp.float32(N))


def _gm(x):
    return pl.pallas_call(
        _gm_body,
        grid=(B,),
        in_specs=[pl.BlockSpec((1, N, E), lambda b: (b, 0, 0))],
        out_specs=pl.BlockSpec((1, 1, E), lambda b: (b, 0, 0)),
        out_shape=jax.ShapeDtypeStruct((B, 1, E), jnp.float32),
    )(x)


_KR = 128


def _knn_body(cq_ref, cqT_ref, idx_ref):
    b = pl.program_id(0)
    ck = cq_ref[0]
    qT = cqT_ref[0]
    dots = lax.dot_general(qT, ck, (((1,), (0,)), ((), ())),
                           precision=lax.Precision.DEFAULT,
                           preferred_element_type=jnp.float32)
    sqk = ck * ck
    k2 = (sqk[0:1] + sqk[1:2]) + sqk[2:3]
    sqq = qT * qT
    q2t = (sqq[:, 0:1] + sqq[:, 1:2]) + sqq[:, 2:3]
    d = (q2t + k2) - 2.0 * dots
    iota = lax.broadcasted_iota(jnp.int32, (_KR, N), 1)
    base = b * N
    for m in range(NH):
        mn = jnp.min(d, axis=1, keepdims=True)
        am = jnp.min(jnp.where(d == mn, iota, N), axis=1, keepdims=True)
        idx_ref[:, m:m + 1] = am + base
        d = jnp.where(iota == am, jnp.float32(jnp.inf), d)


def _knn(cq, cqT):
    return pl.pallas_call(
        _knn_body,
        grid=(B, N // _KR),
        in_specs=[pl.BlockSpec((1, 3, N), lambda b, t: (b, 0, 0)),
                  pl.BlockSpec((1, _KR, 3), lambda b, t: (b, t, 0))],
        out_specs=pl.BlockSpec((_KR, NH), lambda b, t: (b * (N // _KR) + t, 0)),
        out_shape=jax.ShapeDtypeStruct((ROWS, NH), jnp.int32),
    )(cq, cqT)


try:
    _info = plsc.get_sparse_core_info()
    _NC, _NS = _info.num_cores, _info.num_subcores
except Exception:
    _NC, _NS = 2, 16
_NW = _NC * _NS
_GROWS = ROWS * NH
_CH = 128
_NCHUNK = _GROWS // (_NW * _CH)


def _gather_sc(xf, idxf):
    mesh = plsc.VectorSubcoreMesh(core_axis_name="c", subcore_axis_name="s")
    niter = _NCHUNK // 2

    @functools.partial(
        pl.kernel, mesh=mesh,
        out_type=jax.ShapeDtypeStruct((_GROWS, E), jnp.float32),
        scratch_types=[
            pltpu.VMEM((2, _CH), jnp.int32),
            pltpu.VMEM((_CH, E), jnp.float32),
            pltpu.VMEM((_CH, E), jnp.float32),
            pltpu.SemaphoreType.DMA,
            pltpu.SemaphoreType.DMA,
            pltpu.SemaphoreType.DMA,
        ],
    )
    def k(x_hbm, idx_hbm, out_hbm, idx_v, rows0_v, rows1_v, sem0, sem1, semo):
        wid = lax.axis_index("s") * _NC + lax.axis_index("c")
        r0 = wid * _NCHUNK
        pltpu.sync_copy(idx_hbm.at[r0], idx_v.at[0])
        pltpu.async_copy(x_hbm.at[idx_v.at[0]], rows0_v, sem0)

        def body(i, carry):
            r = r0 + 2 * i
            pltpu.sync_copy(idx_hbm.at[r + 1], idx_v.at[1])
            pltpu.async_copy(x_hbm.at[idx_v.at[1]], rows1_v, sem1)
            pltpu.make_async_copy(x_hbm.at[idx_v.at[0]], rows0_v, sem0).wait()
            pltpu.async_copy(rows0_v, out_hbm.at[pl.ds(r * _CH, _CH)],
                             semo).wait()

            @pl.when(i + 1 < niter)
            def _():
                pltpu.sync_copy(idx_hbm.at[r + 2], idx_v.at[0])
                pltpu.async_copy(x_hbm.at[idx_v.at[0]], rows0_v, sem0)

            pltpu.make_async_copy(x_hbm.at[idx_v.at[1]], rows1_v, sem1).wait()
            pltpu.async_copy(rows1_v, out_hbm.at[pl.ds((r + 1) * _CH, _CH)],
                             semo).wait()
            return carry

        lax.fori_loop(0, niter, body, 0)

    return k(xf, idxf)


def _stats_body(v_ref, gm_ref, ld_ref):
    v = v_ref[...]
    s = jnp.sum(v, axis=1)
    m = s / jnp.float32(NH)
    c = v - m[:, None, :]
    ss = jnp.sum(c * c, axis=1)
    st = jnp.sqrt(ss / jnp.float32(NH - 1))
    q = st / gm_ref[0]
    w = q[:, :128] + q[:, 128:]
    acc = w[:, 0:8]
    for mm in range(1, 16):
        acc = acc + w[:, 8 * mm:8 * mm + 8]
    b2 = acc[:, 0:4] + acc[:, 4:8]
    c2 = b2[:, 0:2] + b2[:, 2:4]
    r = c2[:, 0:1] + c2[:, 1:2]
    ld_ref[...] = jnp.broadcast_to(r, (64, 128))


def _stats(xnh, gm):
    return pl.pallas_call(
        _stats_body,
        grid=(ROWS // 64,),
        in_specs=[pl.BlockSpec((64, NH, E), lambda i: (i, 0, 0)),
                  pl.BlockSpec((1, 1, E), lambda i: (i // (N // 64), 0, 0))],
        out_specs=pl.BlockSpec((64, 128), lambda i: (i, 0)),
        out_shape=jax.ShapeDtypeStruct((ROWS, 128), jnp.float32),
    )(xnh, gm)


_RT = 512


def _rank_body(lda_ref, ldb_ref, rank_ref):
    t = pl.program_id(1)
    a = lda_ref[0][:, 0:1]
    bl = ldb_ref[0]
    il = lax.broadcasted_iota(jnp.int32, (_RT, N), 1)
    ir = lax.broadcasted_iota(jnp.int32, (_RT, N), 0) + t * _RT
    gt = (bl > a).astype(jnp.int32)
    eq = ((bl == a) & (il < ir)).astype(jnp.int32)
    cnt = jnp.sum(gt + eq, axis=1, keepdims=True)
    rank_ref[...] = jnp.broadcast_to(cnt, (1, _RT, 128))


def _rank(ldb128, ldrow):
    return pl.pallas_call(
        _rank_body,
        grid=(B, N // _RT),
        in_specs=[pl.BlockSpec((1, _RT, 128), lambda b, t: (b, t, 0)),
                  pl.BlockSpec((1, 1, N), lambda b, t: (b, 0, 0))],
        out_specs=pl.BlockSpec((1, _RT, 128), lambda b, t: (b, t, 0)),
        out_shape=jax.ShapeDtypeStruct((B, N, 128), jnp.int32),
    )(ldb128, ldrow)


_SROW = ROWS // 128
_SPW = _SROW // _NW


def _select_sc(ranks2d, xf, cpad):
    mesh = plsc.VectorSubcoreMesh(core_axis_name="c", subcore_axis_name="s")

    @functools.partial(
        pl.kernel, mesh=mesh,
        out_type=[jax.ShapeDtypeStruct((DUMP + 8, E), jnp.float32),
                  jax.ShapeDtypeStruct((DUMP + 8, 128), jnp.float32)],
        scratch_types=[
            pltpu.VMEM((_SPW, 128), jnp.int32),
            pltpu.VMEM((128,), jnp.int32),
            pltpu.VMEM((128, E), jnp.float32),
            pltpu.VMEM((128, 128), jnp.float32),
            pltpu.SemaphoreType.DMA,
        ],
    )
    def k(rk_hbm, x_hbm, c_hbm, xo_hbm, co_hbm, pos_v, rk_v, xrow_v, crow_v, sem):
        wid = lax.axis_index("s") * _NC + lax.axis_index("c")
        for rl in range(_SPW):
            r = wid * _SPW + rl
            b = r // (N // 128)
            pltpu.sync_copy(rk_hbm.at[r], rk_v)
            for j in range(8):
                rv = rk_v[pl.ds(j * 16, 16)]
                pos = jnp.where(rv < NKEEP, rv + b * NKEEP,
                                jnp.int32(DUMP + j))
                pos_v[rl, pl.ds(j * 16, 16)] = pos
            pltpu.sync_copy(x_hbm.at[pl.ds(r * 128, 128)], xrow_v)
            pltpu.async_copy(xrow_v, xo_hbm.at[pos_v.at[rl]], sem).wait()
            pltpu.sync_copy(c_hbm.at[pl.ds(r * 128, 128)], crow_v)
            pltpu.async_copy(crow_v, co_hbm.at[pos_v.at[rl]], sem).wait()

    return k(ranks2d, xf, cpad)


def kernel(x, coords):
    cq = coords[..., 0]
    cqT = jnp.swapaxes(cq, 1, 2)
    xf = x.reshape(ROWS, E)

    gm = _gm(x)
    idx = _knn(cq, cqT)
    idxf = idx.reshape(_GROWS // 128, 128)
    xnh = _gather_sc(xf, idxf).reshape(ROWS, NH, E)
    ldb = _stats(xnh, gm)
    ld = ldb[:, 0].reshape(B, N)

    ldb128 = ldb.reshape(B, N, 128)
    ldrow = ld.reshape(B, 1, N)
    ranks = _rank(ldb128, ldrow)[:, :, 0]
    ranks2d = ranks.reshape(_SROW, 128)

    cflat = jnp.swapaxes(cq, 1, 2).reshape(ROWS, 3)
    cpad = jnp.concatenate(
        [cflat, jnp.zeros((ROWS, 125), jnp.float32)], axis=1)
    xo_full, co_full = _select_sc(ranks2d, xf, cpad)

    x_out = xo_full[:DUMP].reshape(B, NKEEP, E)
    coords_out = jnp.swapaxes(co_full[:DUMP, :3].reshape(B, NKEEP, 3),
                              1, 2)[..., None]
    return (x_out, coords_out, ld)


# speedup vs baseline: 1.0703x; 1.0703x over previous
import functools

import jax
import jax.numpy as jnp
from jax import lax
from jax.experimental import pallas as pl
from jax.experimental.pallas import tpu as pltpu
from jax.experimental.pallas import tpu_sc as plsc

NH = 32
N = 4096
E = 256
B = 2
NKEEP = 2048
ROWS = B * N
DUMP = B * NKEEP


def _gm_body(x_ref, gm_ref):
    s = jnp.sum(x_ref[...], axis=1, keepdims=True)
    gm_ref[...] = jnp.abs(s / jnp.float32(N))


def _gm(x):
    return pl.pallas_call(
        _gm_body,
        grid=(B,),
        in_specs=[pl.BlockSpec((1, N, E), lambda b: (b, 0, 0))],
        out_specs=pl.BlockSpec((1, 1, E), lambda b: (b, 0, 0)),
        out_shape=jax.ShapeDtypeStruct((B, 1, E), jnp.float32),
    )(x)


_KR = 256


def _knn_body(cq_ref, cqT_ref, idx_ref):
    b = pl.program_id(0)
    ck = cq_ref[0]
    qT = cqT_ref[0]
    dots = lax.dot_general(qT, ck, (((1,), (0,)), ((), ())),
                           precision=lax.Precision.DEFAULT,
                           preferred_element_type=jnp.float32)
    sqk = ck * ck
    k2 = (sqk[0:1] + sqk[1:2]) + sqk[2:3]
    sqq = qT * qT
    q2t = (sqq[:, 0:1] + sqq[:, 1:2]) + sqq[:, 2:3]
    d = (q2t + k2) - 2.0 * dots
    iota = lax.broadcasted_iota(jnp.int32, (_KR, N), 1)
    base = b * N
    for m in range(NH):
        mn = jnp.min(d, axis=1, keepdims=True)
        am = jnp.min(jnp.where(d == mn, iota, N), axis=1, keepdims=True)
        idx_ref[:, m:m + 1] = am + base
        d = jnp.where(iota == am, jnp.float32(jnp.inf), d)


def _knn(cq, cqT):
    return pl.pallas_call(
        _knn_body,
        grid=(B, N // _KR),
        in_specs=[pl.BlockSpec((1, 3, N), lambda b, t: (b, 0, 0)),
                  pl.BlockSpec((1, _KR, 3), lambda b, t: (b, t, 0))],
        out_specs=pl.BlockSpec((_KR, NH), lambda b, t: (b * (N // _KR) + t, 0)),
        out_shape=jax.ShapeDtypeStruct((ROWS, NH), jnp.int32),
    )(cq, cqT)


try:
    _info = plsc.get_sparse_core_info()
    _NC, _NS = _info.num_cores, _info.num_subcores
except Exception:
    _NC, _NS = 2, 16
_NW = _NC * _NS
_GROWS = ROWS * NH
_CH = 128
_NCHUNK = _GROWS // (_NW * _CH)


def _gather_sc(xf, idxf):
    mesh = plsc.VectorSubcoreMesh(core_axis_name="c", subcore_axis_name="s")
    niter = _NCHUNK // 2

    @functools.partial(
        pl.kernel, mesh=mesh,
        out_type=jax.ShapeDtypeStruct((_GROWS, E), jnp.float32),
        scratch_types=[
            pltpu.VMEM((2, _CH), jnp.int32),
            pltpu.VMEM((_CH, E), jnp.float32),
            pltpu.VMEM((_CH, E), jnp.float32),
            pltpu.SemaphoreType.DMA,
            pltpu.SemaphoreType.DMA,
            pltpu.SemaphoreType.DMA,
        ],
    )
    def k(x_hbm, idx_hbm, out_hbm, idx_v, rows0_v, rows1_v, sem0, sem1, semo):
        wid = lax.axis_index("s") * _NC + lax.axis_index("c")
        r0 = wid * _NCHUNK
        pltpu.sync_copy(idx_hbm.at[r0], idx_v.at[0])
        pltpu.async_copy(x_hbm.at[idx_v.at[0]], rows0_v, sem0)

        def body(i, carry):
            r = r0 + 2 * i
            pltpu.sync_copy(idx_hbm.at[r + 1], idx_v.at[1])
            pltpu.async_copy(x_hbm.at[idx_v.at[1]], rows1_v, sem1)
            pltpu.make_async_copy(x_hbm.at[idx_v.at[0]], rows0_v, sem0).wait()
            pltpu.async_copy(rows0_v, out_hbm.at[pl.ds(r * _CH, _CH)],
                             semo).wait()

            @pl.when(i + 1 < niter)
            def _():
                pltpu.sync_copy(idx_hbm.at[r + 2], idx_v.at[0])
                pltpu.async_copy(x_hbm.at[idx_v.at[0]], rows0_v, sem0)

            pltpu.make_async_copy(x_hbm.at[idx_v.at[1]], rows1_v, sem1).wait()
            pltpu.async_copy(rows1_v, out_hbm.at[pl.ds((r + 1) * _CH, _CH)],
                             semo).wait()
            return carry

        lax.fori_loop(0, niter, body, 0)

    return k(xf, idxf)


def _stats_body(v_ref, gm_ref, ld_ref):
    v = v_ref[...]
    s = jnp.sum(v, axis=1)
    m = s / jnp.float32(NH)
    c = v - m[:, None, :]
    ss = jnp.sum(c * c, axis=1)
    st = jnp.sqrt(ss / jnp.float32(NH - 1))
    q = st / gm_ref[0]
    w = q[:, :128] + q[:, 128:]
    acc = w[:, 0:8]
    for mm in range(1, 16):
        acc = acc + w[:, 8 * mm:8 * mm + 8]
    b2 = acc[:, 0:4] + acc[:, 4:8]
    c2 = b2[:, 0:2] + b2[:, 2:4]
    r = c2[:, 0:1] + c2[:, 1:2]
    ld_ref[...] = jnp.broadcast_to(r, (64, 128))


def _stats(xnh, gm):
    return pl.pallas_call(
        _stats_body,
        grid=(ROWS // 64,),
        in_specs=[pl.BlockSpec((64, NH, E), lambda i: (i, 0, 0)),
                  pl.BlockSpec((1, 1, E), lambda i: (i // (N // 64), 0, 0))],
        out_specs=pl.BlockSpec((64, 128), lambda i: (i, 0)),
        out_shape=jax.ShapeDtypeStruct((ROWS, 128), jnp.float32),
    )(xnh, gm)


_RT = 512


def _rank_body(lda_ref, ldb_ref, rank_ref):
    t = pl.program_id(1)
    a = lda_ref[0][:, 0:1]
    bl = ldb_ref[0]
    il = lax.broadcasted_iota(jnp.int32, (_RT, N), 1)
    ir = lax.broadcasted_iota(jnp.int32, (_RT, N), 0) + t * _RT
    gt = (bl > a).astype(jnp.int32)
    eq = ((bl == a) & (il < ir)).astype(jnp.int32)
    cnt = jnp.sum(gt + eq, axis=1, keepdims=True)
    rank_ref[...] = jnp.broadcast_to(cnt, (1, _RT, 128))


def _rank(ldb128, ldrow):
    return pl.pallas_call(
        _rank_body,
        grid=(B, N // _RT),
        in_specs=[pl.BlockSpec((1, _RT, 128), lambda b, t: (b, t, 0)),
                  pl.BlockSpec((1, 1, N), lambda b, t: (b, 0, 0))],
        out_specs=pl.BlockSpec((1, _RT, 128), lambda b, t: (b, t, 0)),
        out_shape=jax.ShapeDtypeStruct((B, N, 128), jnp.int32),
    )(ldb128, ldrow)


_SROW = ROWS // 128
_SPW = _SROW // _NW


def _select_sc(ranks2d, xf, cpad):
    mesh = plsc.VectorSubcoreMesh(core_axis_name="c", subcore_axis_name="s")

    @functools.partial(
        pl.kernel, mesh=mesh,
        out_type=[jax.ShapeDtypeStruct((DUMP + 8, E), jnp.float32),
                  jax.ShapeDtypeStruct((DUMP + 8, 128), jnp.float32)],
        scratch_types=[
            pltpu.VMEM((_SPW, 128), jnp.int32),
            pltpu.VMEM((128,), jnp.int32),
            pltpu.VMEM((128, E), jnp.float32),
            pltpu.VMEM((128, 128), jnp.float32),
            pltpu.SemaphoreType.DMA,
        ],
    )
    def k(rk_hbm, x_hbm, c_hbm, xo_hbm, co_hbm, pos_v, rk_v, xrow_v, crow_v, sem):
        wid = lax.axis_index("s") * _NC + lax.axis_index("c")
        for rl in range(_SPW):
            r = wid * _SPW + rl
            b = r // (N // 128)
            pltpu.sync_copy(rk_hbm.at[r], rk_v)
            for j in range(8):
                rv = rk_v[pl.ds(j * 16, 16)]
                pos = jnp.where(rv < NKEEP, rv + b * NKEEP,
                                jnp.int32(DUMP + j))
                pos_v[rl, pl.ds(j * 16, 16)] = pos
            pltpu.sync_copy(x_hbm.at[pl.ds(r * 128, 128)], xrow_v)
            pltpu.async_copy(xrow_v, xo_hbm.at[pos_v.at[rl]], sem).wait()
            pltpu.sync_copy(c_hbm.at[pl.ds(r * 128, 128)], crow_v)
            pltpu.async_copy(crow_v, co_hbm.at[pos_v.at[rl]], sem).wait()

    return k(ranks2d, xf, cpad)


def kernel(x, coords):
    cq = coords[..., 0]
    cqT = jnp.swapaxes(cq, 1, 2)
    xf = x.reshape(ROWS, E)

    gm = _gm(x)
    idx = _knn(cq, cqT)
    idxf = idx.reshape(_GROWS // 128, 128)
    xnh = _gather_sc(xf, idxf).reshape(ROWS, NH, E)
    ldb = _stats(xnh, gm)
    ld = ldb[:, 0].reshape(B, N)

    ldb128 = ldb.reshape(B, N, 128)
    ldrow = ld.reshape(B, 1, N)
    ranks = _rank(ldb128, ldrow)[:, :, 0]
    ranks2d = ranks.reshape(_SROW, 128)

    cflat = jnp.swapaxes(cq, 1, 2).reshape(ROWS, 3)
    cpad = jnp.concatenate(
        [cflat, jnp.zeros((ROWS, 125), jnp.float32)], axis=1)
    xo_full, co_full = _select_sc(ranks2d, xf, cpad)

    x_out = xo_full[:DUMP].reshape(B, NKEEP, E)
    coords_out = jnp.swapaxes(co_full[:DUMP, :3].reshape(B, NKEEP, 3),
                              1, 2)[..., None]
    return (x_out, coords_out, ld)
